# baseline (speedup 1.0000x reference)
import math
from itertools import permutations as _permutations

import jax
import jax.numpy as jnp
import numpy as np
from jax import lax
from jax.experimental import pallas as pl
from jax.experimental.pallas import tpu as pltpu
from jax.experimental.pallas import tpu_sc as plsc

_L = 8
_S = 4
_D = 3
_ND = 128
_ED = 16
_N = 10000
_P = 24
_PERMS = np.array(list(_permutations(range(_S))), dtype=np.int32)

_M = math.pi / 2

_NPAD = 10240
_WROWS = _NPAD // 32
_GEOM = 256

_BLK = 1000
_G = _N // _BLK

_ATAN_C = (0.9999999227745398, -0.3333223244657235, 0.19974024787565844,
           -0.14047793148813997, 0.10002110154691828, -0.060872867201036907,
           0.02533036269905139, -0.005020633432245819)


def _atan_pos(y):
    big = y > 1.0
    z = jnp.where(big, 1.0 / jnp.maximum(y, 1e-30), y)
    u = z * z
    p = jnp.full_like(u, _ATAN_C[-1])
    for c in _ATAN_C[-2::-1]:
        p = p * u + jnp.float32(c)
    a = z * p
    return jnp.where(big, jnp.float32(_M) - a, a)


def _sc_sqrt(x):
    i = lax.bitcast_convert_type(x, jnp.int32)
    i = jnp.int32(0x5F3759DF) - lax.shift_right_logical(i, 1)
    y = lax.bitcast_convert_type(i, jnp.float32)
    for _ in range(3):
        y = y * (jnp.float32(1.5) - jnp.float32(0.5) * x * y * y)
    return jnp.where(x > 0, x * y, jnp.float32(0.0))


_NGRP = _N // 16
_WG = 20


def _sc_geom_body(pn_hbm, pf_hbm, out_hbm, pnb, pfb, part, sem):
    wid = lax.axis_index("s") * 2 + lax.axis_index("c")
    g0 = (_NGRP * wid) // 32
    g1 = (_NGRP * (wid + 1)) // 32
    d0 = jnp.maximum(g1 - _WG, 0)
    hs = []
    for r in range(12):
        hs.append(pltpu.async_copy(
            pn_hbm.at[pl.ds(r * _N + d0 * 16, _WG * 16)],
            pnb.at[pl.ds(r * _WG * 16, _WG * 16)], sem))
    for r in range(3):
        hs.append(pltpu.async_copy(
            pf_hbm.at[pl.ds(r * _N + d0 * 16, _WG * 16)],
            pfb.at[pl.ds(r * _WG * 16, _WG * 16)], sem))
    zero = jnp.zeros((16,), jnp.float32)
    for j in range(_GEOM // 16):
        part[pl.ds(16 * j, 16)] = zero
    for h in hs:
        h.wait()

    for g in range(_WG):
        gk = d0 + g
        ok = jnp.logical_and(gk >= g0, gk < g1)
        okf = jnp.where(ok, jnp.float32(1.0), jnp.float32(0.0))
        pf_v = [pfb[pl.ds(d * _WG * 16 + 16 * g, 16)] for d in range(3)]
        pe = [pnb[pl.ds((s * 3 + d) * _WG * 16 + 16 * g, 16)] - pf_v[d]
              for s in range(4) for d in range(3)]
        ssq = [pe[3 * s] * pe[3 * s] + pe[3 * s + 1] * pe[3 * s + 1]
               + pe[3 * s + 2] * pe[3 * s + 2] for s in range(4)]
        na = [_sc_sqrt(q) for q in ssq]
        for s in range(4):
            sp = (s - 1) % 4
            dot = (pe[3 * sp] * pe[3 * s]
                   + pe[3 * sp + 1] * pe[3 * s + 1]
                   + pe[3 * sp + 2] * pe[3 * s + 2])
            cosv = dot / jnp.maximum(na[sp] * na[s], jnp.float32(1e-8))
            iv = cosv * okf
            lv = na[s] * okf
            plsc.addupdate(part.at[pl.ds(16 * s, 16)], iv)
            plsc.addupdate(part.at[pl.ds(64 + 16 * s, 16)], lv)
            plsc.addupdate(part.at[pl.ds(128 + 16 * s, 16)], iv * iv)
            plsc.addupdate(part.at[pl.ds(192 + 16 * s, 16)], lv * lv)
    pltpu.sync_copy(part, out_hbm.at[pl.ds(wid * _GEOM, _GEOM)])


def _sc_geom(pn1, pf1):
    mesh = plsc.VectorSubcoreMesh(core_axis_name="c", subcore_axis_name="s")
    f = pl.kernel(
        _sc_geom_body,
        mesh=mesh,
        out_type=jax.ShapeDtypeStruct((32 * _GEOM,), jnp.float32),
        scratch_types=[
            pltpu.VMEM((12 * _WG * 16,), jnp.float32),
            pltpu.VMEM((3 * _WG * 16,), jnp.float32),
            pltpu.VMEM((_GEOM,), jnp.float32),
            pltpu.SemaphoreType.DMA,
        ],
    )
    return f(pn1, pf1)


def _tc_stats_body(xn_ref, xf_ref, ed_ref, out_ref, a_xn, a_xf, a_ed, a_sq):
    i = pl.program_id(0)

    @pl.when(i == 0)
    def _init():
        a_xn[...] = jnp.zeros_like(a_xn)
        a_xf[...] = jnp.zeros_like(a_xf)
        a_ed[...] = jnp.zeros_like(a_ed)
        for k in range(3):
            a_sq[k] = 0.0

    xn = xn_ref[...]
    xf = xf_ref[...]
    ed = ed_ref[...]
    a_xn[...] += jnp.sum(xn, axis=0)
    a_xf[...] += jnp.sum(xf, axis=0, keepdims=True)
    a_ed[...] += jnp.sum(ed, axis=0)
    a_sq[0] = a_sq[0] + jnp.sum(xn * xn)
    a_sq[1] = a_sq[1] + jnp.sum(xf * xf)
    a_sq[2] = a_sq[2] + jnp.sum(ed * ed)

    @pl.when(i == _G - 1)
    def _fin():
        li = lax.broadcasted_iota(jnp.int32, (1, 128), 1)
        srow = (jnp.where(li == 0, a_sq[0], 0.0)
                + jnp.where(li == 1, a_sq[1], 0.0)
                + jnp.where(li == 2, a_sq[2], 0.0))
        zpad = jnp.zeros((1, 112), jnp.float32)
        rows = [a_xn[...], a_xf[...]]
        for s in range(4):
            rows.append(jnp.concatenate([a_ed[s:s + 1, :], zpad], axis=-1))
        rows.append(srow)
        rows.append(jnp.zeros((6, 128), jnp.float32))
        out_ref[...] = jnp.concatenate(rows, axis=0)


def _tc_stats(xn3, xf, ed3):
    return pl.pallas_call(
        _tc_stats_body,
        grid=(_G,),
        in_specs=[
            pl.BlockSpec((_BLK, _S, _ND), lambda i: (i, 0, 0)),
            pl.BlockSpec((_BLK, _ND), lambda i: (i, 0)),
            pl.BlockSpec((_BLK, _S, _ED), lambda i: (i, 0, 0)),
        ],
        out_specs=pl.BlockSpec((16, _ND), lambda i: (0, 0)),
        out_shape=jax.ShapeDtypeStruct((16, _ND), jnp.float32),
        scratch_shapes=[
            pltpu.VMEM((_S, _ND), jnp.float32),
            pltpu.VMEM((1, _ND), jnp.float32),
            pltpu.VMEM((_S, _ED), jnp.float32),
            pltpu.SMEM((4,), jnp.float32),
        ],
    )(xn3, xf, ed3)


def _intra_cols(p12):
    cur = [p12[:, 3 * s:3 * s + 3] for s in range(_S)]
    ssq = [jnp.sum(c * c, axis=-1, keepdims=True) for c in cur]
    na = [jnp.sqrt(q) for q in ssq]
    intra = []
    for s in range(_S):
        sp = (s - 1) % _S
        dot = jnp.sum(cur[sp] * cur[s], axis=-1, keepdims=True)
        intra.append(dot / jnp.maximum(na[sp] * na[s], 1e-8))
    return jnp.concatenate(intra, axis=-1), jnp.concatenate(na, axis=-1)


def _epi_body(st1_ref, st2_ref, pxs0_ref, pxs1_ref, pxs2_ref, pxs3_ref,
              ped0_ref, ped1_ref, ped2_ref, ped3_ref, pps_ref, xc_ref,
              out_ref):
    nf = jnp.float32(_N)
    pxs_refs = (pxs0_ref, pxs1_ref, pxs2_ref, pxs3_ref)
    ped_refs = (ped0_ref, ped1_ref, ped2_ref, ped3_ref)
    st1 = st1_ref[...]
    s_xn = [st1[s:s + 1, :] for s in range(4)]
    s_xf = st1[4:5, :]
    s_ed = [st1[5 + s:6 + s, 0:16] for s in range(4)]
    q_xn = jnp.sum(st1[9:10, 0:1])
    q_xf = jnp.sum(st1[9:10, 1:2])
    q_ed = jnp.sum(st1[9:10, 2:3])
    s2 = jnp.sum(st2_ref[...], axis=0, keepdims=True)
    a_in = [jnp.sum(s2[:, 16 * k:16 * k + 16]) for k in range(4)]
    a_ln = [jnp.sum(s2[:, 64 + 16 * k:80 + 16 * k]) for k in range(4)]
    q_in = jnp.sum(s2[:, 128:192])
    q_ln = jnp.sum(s2[:, 192:256])

    iota = lax.broadcasted_iota(jnp.int32, (_P, 1), 0)
    ot = jnp.zeros((1, _L), jnp.float32)
    oi = lax.broadcasted_iota(jnp.int32, (1, _L), 1)
    for l in range(_L):
        pps = pps_ref[pl.ds(_P * l, _P), :]

        b_in, b_ln = _intra_cols(pps)
        cr_in = sum(b_in[:, k:k + 1] * a_in[k] for k in range(4))
        cr_ln = sum(b_ln[:, k:k + 1] * a_ln[k] for k in range(4))
        t_ang = (q_in - 2.0 * cr_in
                 + nf * jnp.sum(b_in * b_in, -1, keepdims=True))
        t_len = (q_ln - 2.0 * cr_ln
                 + nf * jnp.sum(b_ln * b_ln, -1, keepdims=True))
        t_sup = jnp.full((_P, 1), q_xn)
        t_edg = jnp.full((_P, 1), q_ed)
        for s in range(4):
            pxs = pxs_refs[s][pl.ds(_P * l, _P), :]
            ped = ped_refs[s][pl.ds(_P * l, _P), :]
            t_sup = (t_sup - 2.0 * jnp.sum(pxs * s_xn[s], -1, keepdims=True)
                     + nf * jnp.sum(pxs * pxs, -1, keepdims=True))
            t_edg = (t_edg - 2.0 * jnp.sum(ped * s_ed[s], -1, keepdims=True)
                     + nf * jnp.sum(ped * ped, -1, keepdims=True))

        tmin = jnp.min(t_ang)
        bidx = jnp.min(jnp.where(t_ang <= tmin, iota, _P))
        onehot = iota == bidx
        t_len_b = jnp.sum(jnp.where(onehot, t_len, 0.0))
        t_sup_b = jnp.sum(jnp.where(onehot, t_sup, 0.0))
        t_edg_b = jnp.sum(jnp.where(onehot, t_edg, 0.0))

        xc = xc_ref[pl.ds(l, 1), :]
        t_cen = q_xf - 2.0 * jnp.sum(xc * s_xf) + nf * jnp.sum(xc * xc)

        sc_ang = _atan_pos(1.0 / tmin)
        sc_len = _atan_pos(1.0 / t_len_b)
        sc_sup = _atan_pos(1.0 / t_sup_b)
        sc_cen = _atan_pos(1.0 / t_cen)
        sc_edg = _atan_pos(1.0 / t_edg_b)

        m = jnp.float32(_M)
        tot = ((sc_len - m) ** 2 + (sc_ang - m) ** 2 + (sc_sup - m) ** 2
               + (sc_cen - m) ** 2 + (sc_edg - m) ** 2)
        sc = _atan_pos(1.0 / tot)
        ot = ot + jnp.where(oi == l, sc, 0.0)
    out_ref[...] = ot


def _epilogue(stats1, stats2, pxs_s, ped_s, pps, xc2):
    return pl.pallas_call(
        _epi_body,
        out_shape=jax.ShapeDtypeStruct((1, _L), jnp.float32),
    )(stats1, stats2, *pxs_s, *ped_s, pps, xc2)


def kernel(x_focal, p_focal, x_neighbor, p_neighbor, edge_attr_neighbor,
           x_center, x_support, edge_attr_support, p_support):
    n = x_focal.shape[0]
    pnT = lax.dot_general(jnp.eye(_S * _D, dtype=jnp.float32),
                          p_neighbor.reshape(n, _S * _D),
                          (((1,), (1,)), ((), ()))).reshape(-1)
    pfT = lax.dot_general(jnp.eye(_D, dtype=jnp.float32), p_focal,
                          (((1,), (1,)), ((), ()))).reshape(-1)
    stats2 = _sc_geom(pnT, pfT).reshape(32, _GEOM)

    stats1 = _tc_stats(x_neighbor, x_focal, edge_attr_neighbor)

    pxs_s = [x_support[:, _PERMS[:, s], :].reshape(_L * _P, _ND)
             for s in range(_S)]
    ped_s = [edge_attr_support[:, _PERMS[:, s], :].reshape(_L * _P, _ED)
             for s in range(_S)]
    pps = p_support[:, _PERMS].reshape(_L * _P, _S * _D)
    xc2 = x_center.reshape(_L, _ND)
    out = _epilogue(stats1, stats2, pxs_s, ped_s, pps, xc2)
    return out.reshape(_L)


# speedup vs baseline: 1.8177x; 1.8177x over previous
import math
from itertools import permutations as _permutations

import jax
import jax.numpy as jnp
import numpy as np
from jax import lax
from jax.experimental import pallas as pl
from jax.experimental.pallas import tpu as pltpu
from jax.experimental.pallas import tpu_sc as plsc

_L = 8
_S = 4
_D = 3
_ND = 128
_ED = 16
_N = 10000
_P = 24
_PERMS = np.array(list(_permutations(range(_S))), dtype=np.int32)

_M = math.pi / 2

_NPAD = 10240
_WROWS = _NPAD // 32
_GEOM = 256

_BLK = 1000
_G = _N // _BLK

_ATAN_C = (0.9999999227745398, -0.3333223244657235, 0.19974024787565844,
           -0.14047793148813997, 0.10002110154691828, -0.060872867201036907,
           0.02533036269905139, -0.005020633432245819)


def _atan_pos(y):
    big = y > 1.0
    z = jnp.where(big, 1.0 / jnp.maximum(y, 1e-30), y)
    u = z * z
    p = jnp.full_like(u, _ATAN_C[-1])
    for c in _ATAN_C[-2::-1]:
        p = p * u + jnp.float32(c)
    a = z * p
    return jnp.where(big, jnp.float32(_M) - a, a)


def _sc_sqrt(x):
    i = lax.bitcast_convert_type(x, jnp.int32)
    i = jnp.int32(0x5F3759DF) - lax.shift_right_logical(i, 1)
    y = lax.bitcast_convert_type(i, jnp.float32)
    for _ in range(3):
        y = y * (jnp.float32(1.5) - jnp.float32(0.5) * x * y * y)
    return jnp.where(x > 0, x * y, jnp.float32(0.0))


_NGRP = _N // 16
_WG = 20


def _sc_geom_body(pn_hbm, pf_hbm, out_hbm, pnb, pfb, part, sem):
    wid = lax.axis_index("s") * 2 + lax.axis_index("c")
    g0 = (_NGRP * wid) // 32
    g1 = (_NGRP * (wid + 1)) // 32
    d0 = jnp.maximum(g1 - _WG, 0)
    hs = []
    for r in range(12):
        hs.append(pltpu.async_copy(
            pn_hbm.at[pl.ds(r * _N + d0 * 16, _WG * 16)],
            pnb.at[pl.ds(r * _WG * 16, _WG * 16)], sem))
    for r in range(3):
        hs.append(pltpu.async_copy(
            pf_hbm.at[pl.ds(r * _N + d0 * 16, _WG * 16)],
            pfb.at[pl.ds(r * _WG * 16, _WG * 16)], sem))
    zero = jnp.zeros((16,), jnp.float32)
    for j in range(_GEOM // 16):
        part[pl.ds(16 * j, 16)] = zero
    for h in hs:
        h.wait()

    for g in range(_WG):
        gk = d0 + g
        ok = jnp.logical_and(gk >= g0, gk < g1)
        okf = jnp.where(ok, jnp.float32(1.0), jnp.float32(0.0))
        pf_v = [pfb[pl.ds(d * _WG * 16 + 16 * g, 16)] for d in range(3)]
        pe = [pnb[pl.ds((s * 3 + d) * _WG * 16 + 16 * g, 16)] - pf_v[d]
              for s in range(4) for d in range(3)]
        ssq = [pe[3 * s] * pe[3 * s] + pe[3 * s + 1] * pe[3 * s + 1]
               + pe[3 * s + 2] * pe[3 * s + 2] for s in range(4)]
        na = [_sc_sqrt(q) for q in ssq]
        for s in range(4):
            sp = (s - 1) % 4
            dot = (pe[3 * sp] * pe[3 * s]
                   + pe[3 * sp + 1] * pe[3 * s + 1]
                   + pe[3 * sp + 2] * pe[3 * s + 2])
            cosv = dot / jnp.maximum(na[sp] * na[s], jnp.float32(1e-8))
            iv = cosv * okf
            lv = na[s] * okf
            plsc.addupdate(part.at[pl.ds(16 * s, 16)], iv)
            plsc.addupdate(part.at[pl.ds(64 + 16 * s, 16)], lv)
            plsc.addupdate(part.at[pl.ds(128 + 16 * s, 16)], iv * iv)
            plsc.addupdate(part.at[pl.ds(192 + 16 * s, 16)], lv * lv)
    pltpu.sync_copy(part, out_hbm.at[pl.ds(wid * _GEOM, _GEOM)])


def _sc_geom(pn1, pf1):
    mesh = plsc.VectorSubcoreMesh(core_axis_name="c", subcore_axis_name="s")
    f = pl.kernel(
        _sc_geom_body,
        mesh=mesh,
        out_type=jax.ShapeDtypeStruct((32 * _GEOM,), jnp.float32),
        scratch_types=[
            pltpu.VMEM((12 * _WG * 16,), jnp.float32),
            pltpu.VMEM((3 * _WG * 16,), jnp.float32),
            pltpu.VMEM((_GEOM,), jnp.float32),
            pltpu.SemaphoreType.DMA,
        ],
    )
    return f(pn1, pf1)


def _tc_stats_body(xn_ref, xf_ref, ed_ref, out_ref, a_xn, a_xf, a_sq):
    i = pl.program_id(0)

    @pl.when(i == 0)
    def _init():
        a_xn[...] = jnp.zeros_like(a_xn)
        a_xf[...] = jnp.zeros_like(a_xf)
        for k in range(3):
            a_sq[k] = 0.0

    xn = xn_ref[...]
    xf = xf_ref[...]
    a_xn[...] += jnp.sum(xn, axis=0)
    a_xf[...] += jnp.sum(xf, axis=0, keepdims=True)
    a_sq[0] = a_sq[0] + jnp.sum(xn * xn)
    a_sq[1] = a_sq[1] + jnp.sum(xf * xf)

    @pl.when(i == _G - 1)
    def _fin():
        ed = ed_ref[...]
        a_ed = jnp.sum(ed, axis=2)
        q_ed = jnp.sum(ed * ed)
        li = lax.broadcasted_iota(jnp.int32, (1, 128), 1)
        srow = (jnp.where(li == 0, a_sq[0], 0.0)
                + jnp.where(li == 1, a_sq[1], 0.0)
                + jnp.where(li == 2, q_ed, 0.0))
        zpad = jnp.zeros((1, 112), jnp.float32)
        rows = [a_xn[...], a_xf[...]]
        for s in range(4):
            rows.append(jnp.concatenate([a_ed[s:s + 1, :], zpad], axis=-1))
        rows.append(srow)
        rows.append(jnp.zeros((6, 128), jnp.float32))
        out_ref[...] = jnp.concatenate(rows, axis=0)


def _tc_stats(xn3, xf, edT):
    return pl.pallas_call(
        _tc_stats_body,
        grid=(_G,),
        in_specs=[
            pl.BlockSpec((_BLK, _S, _ND), lambda i: (i, 0, 0)),
            pl.BlockSpec((_BLK, _ND), lambda i: (i, 0)),
            pl.BlockSpec((_S, _ED, _N), lambda i: (0, 0, 0)),
        ],
        out_specs=pl.BlockSpec((16, _ND), lambda i: (0, 0)),
        out_shape=jax.ShapeDtypeStruct((16, _ND), jnp.float32),
        scratch_shapes=[
            pltpu.VMEM((_S, _ND), jnp.float32),
            pltpu.VMEM((1, _ND), jnp.float32),
            pltpu.SMEM((4,), jnp.float32),
        ],
    )(xn3, xf, edT)


def _intra_cols(p12):
    cur = [p12[:, 3 * s:3 * s + 3] for s in range(_S)]
    ssq = [jnp.sum(c * c, axis=-1, keepdims=True) for c in cur]
    na = [jnp.sqrt(q) for q in ssq]
    intra = []
    for s in range(_S):
        sp = (s - 1) % _S
        dot = jnp.sum(cur[sp] * cur[s], axis=-1, keepdims=True)
        intra.append(dot / jnp.maximum(na[sp] * na[s], 1e-8))
    return jnp.concatenate(intra, axis=-1), jnp.concatenate(na, axis=-1)


_OH = [np.asarray(_PERMS[:, s][:, None] == np.arange(_S)[None, :],
                  dtype=np.float32) for s in range(_S)]


def _epi_body(st1_ref, st2_ref, xs_ref, es_ref, ps_ref, xc_ref, oh_ref,
              out_ref):
    nf = jnp.float32(_N)
    st1 = st1_ref[...]
    s_xn = [st1[s:s + 1, :] for s in range(4)]
    s_xf = st1[4:5, :]
    s_ed = [st1[5 + s:6 + s, 0:16] for s in range(4)]
    q_xn = jnp.sum(st1[9:10, 0:1])
    q_xf = jnp.sum(st1[9:10, 1:2])
    q_ed = jnp.sum(st1[9:10, 2:3])
    s2 = jnp.sum(st2_ref[...], axis=0, keepdims=True)
    a_in = [jnp.sum(s2[:, 16 * k:16 * k + 16]) for k in range(4)]
    a_ln = [jnp.sum(s2[:, 64 + 16 * k:80 + 16 * k]) for k in range(4)]
    q_in = jnp.sum(s2[:, 128:192])
    q_ln = jnp.sum(s2[:, 192:256])

    xs = xs_ref[...]
    es = es_ref[...]
    ps = ps_ref[...]
    ohv = oh_ref[...]
    oh = [ohv[s] for s in range(4)]
    dn = (((1,), (0,)), ((), ()))

    iota = lax.broadcasted_iota(jnp.int32, (_P, 1), 0)
    ot = jnp.zeros((1, _L), jnp.float32)
    oi = lax.broadcasted_iota(jnp.int32, (1, _L), 1)
    for l in range(_L):
        ps_l = ps[l]
        pps = jnp.concatenate(
            [lax.dot_general(oh[s], ps_l, dn) for s in range(4)],
            axis=-1)

        b_in, b_ln = _intra_cols(pps)
        cr_in = sum(b_in[:, k:k + 1] * a_in[k] for k in range(4))
        cr_ln = sum(b_ln[:, k:k + 1] * a_ln[k] for k in range(4))
        t_ang = (q_in - 2.0 * cr_in
                 + nf * jnp.sum(b_in * b_in, -1, keepdims=True))
        t_len = (q_ln - 2.0 * cr_ln
                 + nf * jnp.sum(b_ln * b_ln, -1, keepdims=True))
        t_sup = jnp.full((_P, 1), q_xn)
        t_edg = jnp.full((_P, 1), q_ed)
        xs_l = xs[l]
        es_l = es[l]
        for s in range(4):
            pxs = lax.dot_general(oh[s], xs_l, dn)
            ped = lax.dot_general(oh[s], es_l, dn)
            t_sup = (t_sup - 2.0 * jnp.sum(pxs * s_xn[s], -1, keepdims=True)
                     + nf * jnp.sum(pxs * pxs, -1, keepdims=True))
            t_edg = (t_edg - 2.0 * jnp.sum(ped * s_ed[s], -1, keepdims=True)
                     + nf * jnp.sum(ped * ped, -1, keepdims=True))

        tmin = jnp.min(t_ang)
        bidx = jnp.min(jnp.where(t_ang <= tmin, iota, _P))
        onehot = iota == bidx
        t_len_b = jnp.sum(jnp.where(onehot, t_len, 0.0))
        t_sup_b = jnp.sum(jnp.where(onehot, t_sup, 0.0))
        t_edg_b = jnp.sum(jnp.where(onehot, t_edg, 0.0))

        xc = xc_ref[pl.ds(l, 1), :]
        t_cen = q_xf - 2.0 * jnp.sum(xc * s_xf) + nf * jnp.sum(xc * xc)

        sc_ang = _atan_pos(1.0 / tmin)
        sc_len = _atan_pos(1.0 / t_len_b)
        sc_sup = _atan_pos(1.0 / t_sup_b)
        sc_cen = _atan_pos(1.0 / t_cen)
        sc_edg = _atan_pos(1.0 / t_edg_b)

        m = jnp.float32(_M)
        tot = ((sc_len - m) ** 2 + (sc_ang - m) ** 2 + (sc_sup - m) ** 2
               + (sc_cen - m) ** 2 + (sc_edg - m) ** 2)
        sc = _atan_pos(1.0 / tot)
        ot = ot + jnp.where(oi == l, sc, 0.0)
    out_ref[...] = ot


def _epilogue(stats1, stats2, xs, es, ps, xc2):
    return pl.pallas_call(
        _epi_body,
        out_shape=jax.ShapeDtypeStruct((1, _L), jnp.float32),
    )(stats1, stats2, xs, es, ps, xc2, jnp.asarray(np.stack(_OH)))


def kernel(x_focal, p_focal, x_neighbor, p_neighbor, edge_attr_neighbor,
           x_center, x_support, edge_attr_support, p_support):
    n = x_focal.shape[0]
    pnT = lax.dot_general(jnp.eye(_S * _D, dtype=jnp.float32),
                          p_neighbor.reshape(n, _S * _D),
                          (((1,), (1,)), ((), ()))).reshape(-1)
    pfT = lax.dot_general(jnp.eye(_D, dtype=jnp.float32), p_focal,
                          (((1,), (1,)), ((), ()))).reshape(-1)
    stats2 = _sc_geom(pnT, pfT).reshape(32, _GEOM)

    stats1 = _tc_stats(x_neighbor, x_focal,
                       edge_attr_neighbor.transpose(1, 2, 0))

    xc2 = x_center.reshape(_L, _ND)
    out = _epilogue(stats1, stats2, x_support, edge_attr_support, p_support,
                    xc2)
    return out.reshape(_L)


# speedup vs baseline: 1.8476x; 1.0164x over previous
import math
from itertools import permutations as _permutations

import jax
import jax.numpy as jnp
import numpy as np
from jax import lax
from jax.experimental import pallas as pl
from jax.experimental.pallas import tpu as pltpu
from jax.experimental.pallas import tpu_sc as plsc

_L = 8
_S = 4
_D = 3
_ND = 128
_ED = 16
_N = 10000
_P = 24
_PERMS = np.array(list(_permutations(range(_S))), dtype=np.int32)

_M = math.pi / 2

_NPAD = 10240
_WROWS = _NPAD // 32
_GEOM = 256

_BLK = 1000
_G = _N // _BLK

_ATAN_C = (0.9999999227745398, -0.3333223244657235, 0.19974024787565844,
           -0.14047793148813997, 0.10002110154691828, -0.060872867201036907,
           0.02533036269905139, -0.005020633432245819)


def _atan_pos(y):
    big = y > 1.0
    z = jnp.where(big, 1.0 / jnp.maximum(y, 1e-30), y)
    u = z * z
    p = jnp.full_like(u, _ATAN_C[-1])
    for c in _ATAN_C[-2::-1]:
        p = p * u + jnp.float32(c)
    a = z * p
    return jnp.where(big, jnp.float32(_M) - a, a)


def _sc_sqrt(x):
    i = lax.bitcast_convert_type(x, jnp.int32)
    i = jnp.int32(0x5F3759DF) - lax.shift_right_logical(i, 1)
    y = lax.bitcast_convert_type(i, jnp.float32)
    for _ in range(3):
        y = y * (jnp.float32(1.5) - jnp.float32(0.5) * x * y * y)
    return jnp.where(x > 0, x * y, jnp.float32(0.0))


_NGRP = _N // 16
_NWRK = 16
_WG = 40


def _sc_geom_body(pn_hbm, pf_hbm, out_hbm, pnb, pfb, part, sem):
    wid = lax.axis_index("s")
    g0 = (_NGRP * wid) // _NWRK
    g1 = (_NGRP * (wid + 1)) // _NWRK
    d0 = jnp.maximum(g1 - _WG, 0)
    hs = []
    for r in range(12):
        hs.append(pltpu.async_copy(
            pn_hbm.at[pl.ds(r * _N + d0 * 16, _WG * 16)],
            pnb.at[pl.ds(r * _WG * 16, _WG * 16)], sem))
    for r in range(3):
        hs.append(pltpu.async_copy(
            pf_hbm.at[pl.ds(r * _N + d0 * 16, _WG * 16)],
            pfb.at[pl.ds(r * _WG * 16, _WG * 16)], sem))
    zero = jnp.zeros((16,), jnp.float32)
    for j in range(_GEOM // 16):
        part[pl.ds(16 * j, 16)] = zero
    for h in hs:
        h.wait()

    for g in range(_WG):
        gk = d0 + g
        ok = jnp.logical_and(gk >= g0, gk < g1)
        okf = jnp.where(ok, jnp.float32(1.0), jnp.float32(0.0))
        pf_v = [pfb[pl.ds(d * _WG * 16 + 16 * g, 16)] for d in range(3)]
        pe = [pnb[pl.ds((s * 3 + d) * _WG * 16 + 16 * g, 16)] - pf_v[d]
              for s in range(4) for d in range(3)]
        ssq = [pe[3 * s] * pe[3 * s] + pe[3 * s + 1] * pe[3 * s + 1]
               + pe[3 * s + 2] * pe[3 * s + 2] for s in range(4)]
        na = [_sc_sqrt(q) for q in ssq]
        for s in range(4):
            sp = (s - 1) % 4
            dot = (pe[3 * sp] * pe[3 * s]
                   + pe[3 * sp + 1] * pe[3 * s + 1]
                   + pe[3 * sp + 2] * pe[3 * s + 2])
            cosv = dot / jnp.maximum(na[sp] * na[s], jnp.float32(1e-8))
            iv = cosv * okf
            lv = na[s] * okf
            plsc.addupdate(part.at[pl.ds(16 * s, 16)], iv)
            plsc.addupdate(part.at[pl.ds(64 + 16 * s, 16)], lv)
            plsc.addupdate(part.at[pl.ds(128 + 16 * s, 16)], iv * iv)
            plsc.addupdate(part.at[pl.ds(192 + 16 * s, 16)], lv * lv)
    pltpu.sync_copy(part, out_hbm.at[pl.ds(wid * _GEOM, _GEOM)])


def _sc_geom(pn1, pf1):
    mesh = plsc.VectorSubcoreMesh(core_axis_name="c", subcore_axis_name="s",
                                  num_cores=1)
    f = pl.kernel(
        _sc_geom_body,
        mesh=mesh,
        out_type=jax.ShapeDtypeStruct((_NWRK * _GEOM,), jnp.float32),
        scratch_types=[
            pltpu.VMEM((12 * _WG * 16,), jnp.float32),
            pltpu.VMEM((3 * _WG * 16,), jnp.float32),
            pltpu.VMEM((_GEOM,), jnp.float32),
            pltpu.SemaphoreType.DMA,
        ],
    )
    return f(pn1, pf1)


def _tc_stats_body(xn_ref, xf_ref, ed_ref, out_ref, a_xn, a_xnq, a_xf,
                   a_xfq):
    i = pl.program_id(0)

    @pl.when(i == 0)
    def _init():
        a_xn[...] = jnp.zeros_like(a_xn)
        a_xnq[...] = jnp.zeros_like(a_xnq)
        a_xf[...] = jnp.zeros_like(a_xf)
        a_xfq[...] = jnp.zeros_like(a_xfq)

    xn = xn_ref[...]
    xf = xf_ref[...]
    ones = jnp.ones((1, _BLK), jnp.float32)
    dnr = (((1,), (0,)), ((), ()))
    a_xn[...] += lax.dot_general(ones, xn, dnr)[0]
    a_xnq[...] += lax.dot_general(ones, xn * xn, dnr)[0]
    a_xf[...] += lax.dot_general(ones, xf, dnr)
    a_xfq[...] += lax.dot_general(ones, xf * xf, dnr)

    @pl.when(i == _G - 1)
    def _fin():
        ed = ed_ref[...]
        a_ed = jnp.sum(ed, axis=2)
        q_ed = jnp.sum(ed * ed)
        li = lax.broadcasted_iota(jnp.int32, (1, 128), 1)
        srow = (jnp.where(li == 0, jnp.sum(a_xnq[...]), 0.0)
                + jnp.where(li == 1, jnp.sum(a_xfq[...]), 0.0)
                + jnp.where(li == 2, q_ed, 0.0))
        zpad = jnp.zeros((1, 112), jnp.float32)
        rows = [a_xn[...], a_xf[...]]
        for s in range(4):
            rows.append(jnp.concatenate([a_ed[s:s + 1, :], zpad], axis=-1))
        rows.append(srow)
        rows.append(jnp.zeros((6, 128), jnp.float32))
        out_ref[...] = jnp.concatenate(rows, axis=0)


def _tc_stats(xn3, xf, edT):
    return pl.pallas_call(
        _tc_stats_body,
        grid=(_G,),
        in_specs=[
            pl.BlockSpec((_BLK, _S, _ND), lambda i: (i, 0, 0)),
            pl.BlockSpec((_BLK, _ND), lambda i: (i, 0)),
            pl.BlockSpec((_S, _ED, _N), lambda i: (0, 0, 0)),
        ],
        out_specs=pl.BlockSpec((16, _ND), lambda i: (0, 0)),
        out_shape=jax.ShapeDtypeStruct((16, _ND), jnp.float32),
        scratch_shapes=[
            pltpu.VMEM((_S, _ND), jnp.float32),
            pltpu.VMEM((_S, _ND), jnp.float32),
            pltpu.VMEM((1, _ND), jnp.float32),
            pltpu.VMEM((1, _ND), jnp.float32),
        ],
    )(xn3, xf, edT)


def _intra_cols(p12):
    cur = [p12[:, 3 * s:3 * s + 3] for s in range(_S)]
    ssq = [jnp.sum(c * c, axis=-1, keepdims=True) for c in cur]
    na = [jnp.sqrt(q) for q in ssq]
    intra = []
    for s in range(_S):
        sp = (s - 1) % _S
        dot = jnp.sum(cur[sp] * cur[s], axis=-1, keepdims=True)
        intra.append(dot / jnp.maximum(na[sp] * na[s], 1e-8))
    return jnp.concatenate(intra, axis=-1), jnp.concatenate(na, axis=-1)


_OHB = np.zeros((_S, _L * _P, _L * _S), np.float32)
for _s in range(_S):
    for _l in range(_L):
        for _p in range(_P):
            _OHB[_s, _l * _P + _p, _l * _S + _PERMS[_p, _s]] = 1.0


def _epi_body(st1_ref, st2_ref, xs_ref, es_ref, ps_ref, xc_ref, oh_ref,
              out_ref):
    nf = jnp.float32(_N)
    st1 = st1_ref[...]
    s_xn = [st1[s:s + 1, :] for s in range(4)]
    s_xf = st1[4:5, :]
    s_ed = [st1[5 + s:6 + s, 0:16] for s in range(4)]
    q_xn = jnp.sum(st1[9:10, 0:1])
    q_xf = jnp.sum(st1[9:10, 1:2])
    q_ed = jnp.sum(st1[9:10, 2:3])
    s2 = jnp.sum(st2_ref[...], axis=0, keepdims=True)
    a_in = [jnp.sum(s2[:, 16 * k:16 * k + 16]) for k in range(4)]
    a_ln = [jnp.sum(s2[:, 64 + 16 * k:80 + 16 * k]) for k in range(4)]
    q_in = jnp.sum(s2[:, 128:192])
    q_ln = jnp.sum(s2[:, 192:256])

    xs = xs_ref[...]
    es = es_ref[...]
    ps = ps_ref[...]
    xc = xc_ref[...]
    ohv = oh_ref[...]
    dn = (((1,), (0,)), ((), ()))

    pps = jnp.concatenate(
        [lax.dot_general(ohv[s], ps, dn) for s in range(4)],
        axis=-1)

    b_in, b_ln = _intra_cols(pps)
    cr_in = sum(b_in[:, k:k + 1] * a_in[k] for k in range(4))
    cr_ln = sum(b_ln[:, k:k + 1] * a_ln[k] for k in range(4))
    t_ang = (q_in - 2.0 * cr_in
             + nf * jnp.sum(b_in * b_in, -1, keepdims=True))
    t_len = (q_ln - 2.0 * cr_ln
             + nf * jnp.sum(b_ln * b_ln, -1, keepdims=True))
    t_sup = jnp.full((_L * _P, 1), q_xn)
    t_edg = jnp.full((_L * _P, 1), q_ed)
    for s in range(4):
        pxs = lax.dot_general(ohv[s], xs, dn)
        ped = lax.dot_general(ohv[s], es, dn)
        t_sup = (t_sup - 2.0 * jnp.sum(pxs * s_xn[s], -1, keepdims=True)
                 + nf * jnp.sum(pxs * pxs, -1, keepdims=True))
        t_edg = (t_edg - 2.0 * jnp.sum(ped * s_ed[s], -1, keepdims=True)
                 + nf * jnp.sum(ped * ped, -1, keepdims=True))

    t_cen8 = (q_xf - 2.0 * jnp.sum(xc * s_xf, -1, keepdims=True)
              + nf * jnp.sum(xc * xc, -1, keepdims=True))

    iota = lax.broadcasted_iota(jnp.int32, (_L * _P, 1), 0)
    lid = iota // _P
    i8 = lax.broadcasted_iota(jnp.int32, (_L, 1), 0)
    big = jnp.float32(3.0e38)
    ot = jnp.zeros((1, _L), jnp.float32)
    oi = lax.broadcasted_iota(jnp.int32, (1, _L), 1)
    for l in range(_L):
        ml = lid == l
        tmin = jnp.min(jnp.where(ml, t_ang, big))
        bidx = jnp.min(jnp.where(ml & (t_ang <= tmin), iota, _L * _P))
        onehot = iota == bidx
        t_len_b = jnp.sum(jnp.where(onehot, t_len, 0.0))
        t_sup_b = jnp.sum(jnp.where(onehot, t_sup, 0.0))
        t_edg_b = jnp.sum(jnp.where(onehot, t_edg, 0.0))
        t_cen = jnp.sum(jnp.where(i8 == l, t_cen8, 0.0))

        sc_ang = _atan_pos(1.0 / tmin)
        sc_len = _atan_pos(1.0 / t_len_b)
        sc_sup = _atan_pos(1.0 / t_sup_b)
        sc_cen = _atan_pos(1.0 / t_cen)
        sc_edg = _atan_pos(1.0 / t_edg_b)

        m = jnp.float32(_M)
        tot = ((sc_len - m) ** 2 + (sc_ang - m) ** 2 + (sc_sup - m) ** 2
               + (sc_cen - m) ** 2 + (sc_edg - m) ** 2)
        sc = _atan_pos(1.0 / tot)
        ot = ot + jnp.where(oi == l, sc, 0.0)
    out_ref[...] = ot


def _epilogue(stats1, stats2, xs, es, ps, xc2):
    return pl.pallas_call(
        _epi_body,
        out_shape=jax.ShapeDtypeStruct((1, _L), jnp.float32),
    )(stats1, stats2, xs, es, ps, xc2, jnp.asarray(_OHB))


def kernel(x_focal, p_focal, x_neighbor, p_neighbor, edge_attr_neighbor,
           x_center, x_support, edge_attr_support, p_support):
    n = x_focal.shape[0]
    pnT = lax.dot_general(jnp.eye(_S * _D, dtype=jnp.float32),
                          p_neighbor.reshape(n, _S * _D),
                          (((1,), (1,)), ((), ()))).reshape(-1)
    pfT = lax.dot_general(jnp.eye(_D, dtype=jnp.float32), p_focal,
                          (((1,), (1,)), ((), ()))).reshape(-1)
    stats2 = _sc_geom(pnT, pfT).reshape(_NWRK, _GEOM)

    stats1 = _tc_stats(x_neighbor, x_focal,
                       edge_attr_neighbor.transpose(1, 2, 0))

    xc2 = x_center.reshape(_L, _ND)
    out = _epilogue(stats1, stats2,
                    x_support.reshape(_L * _S, _ND),
                    edge_attr_support.reshape(_L * _S, _ED),
                    p_support.reshape(_L * _S, _D), xc2)
    return out.reshape(_L)


# speedup vs baseline: 1.9356x; 1.0477x over previous
import math
from itertools import permutations as _permutations

import jax
import jax.numpy as jnp
import numpy as np
from jax import lax
from jax.experimental import pallas as pl
from jax.experimental.pallas import tpu as pltpu
from jax.experimental.pallas import tpu_sc as plsc

_L = 8
_S = 4
_D = 3
_ND = 128
_ED = 16
_N = 10000
_P = 24
_PERMS = np.array(list(_permutations(range(_S))), dtype=np.int32)

_M = math.pi / 2

_NPAD = 10240
_WROWS = _NPAD // 32
_GEOM = 256

_BLK = 2000
_G = _N // _BLK

_ATAN_C = (0.9999999227745398, -0.3333223244657235, 0.19974024787565844,
           -0.14047793148813997, 0.10002110154691828, -0.060872867201036907,
           0.02533036269905139, -0.005020633432245819)


def _atan_pos(y):
    big = y > 1.0
    z = jnp.where(big, 1.0 / jnp.maximum(y, 1e-30), y)
    u = z * z
    p = jnp.full_like(u, _ATAN_C[-1])
    for c in _ATAN_C[-2::-1]:
        p = p * u + jnp.float32(c)
    a = z * p
    return jnp.where(big, jnp.float32(_M) - a, a)


def _sc_sqrt(x):
    i = lax.bitcast_convert_type(x, jnp.int32)
    i = jnp.int32(0x5F3759DF) - lax.shift_right_logical(i, 1)
    y = lax.bitcast_convert_type(i, jnp.float32)
    for _ in range(3):
        y = y * (jnp.float32(1.5) - jnp.float32(0.5) * x * y * y)
    return jnp.where(x > 0, x * y, jnp.float32(0.0))


_NGRP = _N // 16
_NWRK = 16
_WG = 40


def _sc_geom_body(pn_hbm, pf_hbm, out_hbm, pnb, pfb, part, sem):
    wid = lax.axis_index("s")
    g0 = (_NGRP * wid) // _NWRK
    g1 = (_NGRP * (wid + 1)) // _NWRK
    d0 = jnp.maximum(g1 - _WG, 0)
    hs = []
    for r in range(12):
        hs.append(pltpu.async_copy(
            pn_hbm.at[pl.ds(r * _N + d0 * 16, _WG * 16)],
            pnb.at[pl.ds(r * _WG * 16, _WG * 16)], sem))
    for r in range(3):
        hs.append(pltpu.async_copy(
            pf_hbm.at[pl.ds(r * _N + d0 * 16, _WG * 16)],
            pfb.at[pl.ds(r * _WG * 16, _WG * 16)], sem))
    zero = jnp.zeros((16,), jnp.float32)
    acc = [[zero] * 4 for _ in range(4)]
    for h in hs:
        h.wait()

    for g in range(_WG):
        gk = d0 + g
        ok = jnp.logical_and(gk >= g0, gk < g1)
        okf = jnp.where(ok, jnp.float32(1.0), jnp.float32(0.0))
        pf_v = [pfb[pl.ds(d * _WG * 16 + 16 * g, 16)] for d in range(3)]
        pe = [pnb[pl.ds((s * 3 + d) * _WG * 16 + 16 * g, 16)] - pf_v[d]
              for s in range(4) for d in range(3)]
        ssq = [pe[3 * s] * pe[3 * s] + pe[3 * s + 1] * pe[3 * s + 1]
               + pe[3 * s + 2] * pe[3 * s + 2] for s in range(4)]
        na = [_sc_sqrt(q) for q in ssq]
        for s in range(4):
            sp = (s - 1) % 4
            dot = (pe[3 * sp] * pe[3 * s]
                   + pe[3 * sp + 1] * pe[3 * s + 1]
                   + pe[3 * sp + 2] * pe[3 * s + 2])
            cosv = dot / jnp.maximum(na[sp] * na[s], jnp.float32(1e-8))
            iv = cosv * okf
            lv = na[s] * okf
            acc[0][s] = acc[0][s] + iv
            acc[1][s] = acc[1][s] + lv
            acc[2][s] = acc[2][s] + iv * iv
            acc[3][s] = acc[3][s] + lv * lv
    for k in range(4):
        for s in range(4):
            part[pl.ds(64 * k + 16 * s, 16)] = acc[k][s]
    pltpu.sync_copy(part, out_hbm.at[pl.ds(wid * _GEOM, _GEOM)])


def _sc_geom(pn1, pf1):
    mesh = plsc.VectorSubcoreMesh(core_axis_name="c", subcore_axis_name="s",
                                  num_cores=1)
    f = pl.kernel(
        _sc_geom_body,
        mesh=mesh,
        out_type=jax.ShapeDtypeStruct((_NWRK * _GEOM,), jnp.float32),
        scratch_types=[
            pltpu.VMEM((12 * _WG * 16,), jnp.float32),
            pltpu.VMEM((3 * _WG * 16,), jnp.float32),
            pltpu.VMEM((_GEOM,), jnp.float32),
            pltpu.SemaphoreType.DMA,
        ],
    )
    return f(pn1, pf1)


def _tc_stats_body(xn_ref, xf_ref, ed_ref, out_ref, a_xn, a_xnq, a_xf,
                   a_xfq):
    i = pl.program_id(0)

    @pl.when(i == 0)
    def _init():
        a_xn[...] = jnp.zeros_like(a_xn)
        a_xnq[...] = jnp.zeros_like(a_xnq)
        a_xf[...] = jnp.zeros_like(a_xf)
        a_xfq[...] = jnp.zeros_like(a_xfq)

    xn = xn_ref[...]
    xf = xf_ref[...]
    ones = jnp.ones((1, _BLK), jnp.float32)
    dnr = (((1,), (0,)), ((), ()))
    a_xn[...] += lax.dot_general(ones, xn, dnr)[0]
    a_xnq[...] += lax.dot_general(ones, xn * xn, dnr)[0]
    a_xf[...] += lax.dot_general(ones, xf, dnr)
    a_xfq[...] += lax.dot_general(ones, xf * xf, dnr)

    @pl.when(i == _G - 1)
    def _fin():
        ed = ed_ref[...]
        a_ed = jnp.sum(ed, axis=2)
        q_ed = jnp.sum(ed * ed)
        li = lax.broadcasted_iota(jnp.int32, (1, 128), 1)
        srow = (jnp.where(li == 0, jnp.sum(a_xnq[...]), 0.0)
                + jnp.where(li == 1, jnp.sum(a_xfq[...]), 0.0)
                + jnp.where(li == 2, q_ed, 0.0))
        zpad = jnp.zeros((1, 112), jnp.float32)
        rows = [a_xn[...], a_xf[...]]
        for s in range(4):
            rows.append(jnp.concatenate([a_ed[s:s + 1, :], zpad], axis=-1))
        rows.append(srow)
        rows.append(jnp.zeros((6, 128), jnp.float32))
        out_ref[...] = jnp.concatenate(rows, axis=0)


def _tc_stats(xn3, xf, edT):
    return pl.pallas_call(
        _tc_stats_body,
        grid=(_G,),
        in_specs=[
            pl.BlockSpec((_BLK, _S, _ND), lambda i: (i, 0, 0)),
            pl.BlockSpec((_BLK, _ND), lambda i: (i, 0)),
            pl.BlockSpec((_S, _ED, _N), lambda i: (0, 0, 0)),
        ],
        out_specs=pl.BlockSpec((16, _ND), lambda i: (0, 0)),
        out_shape=jax.ShapeDtypeStruct((16, _ND), jnp.float32),
        scratch_shapes=[
            pltpu.VMEM((_S, _ND), jnp.float32),
            pltpu.VMEM((_S, _ND), jnp.float32),
            pltpu.VMEM((1, _ND), jnp.float32),
            pltpu.VMEM((1, _ND), jnp.float32),
        ],
    )(xn3, xf, edT)


def _intra_cols(p12):
    cur = [p12[:, 3 * s:3 * s + 3] for s in range(_S)]
    ssq = [jnp.sum(c * c, axis=-1, keepdims=True) for c in cur]
    na = [jnp.sqrt(q) for q in ssq]
    intra = []
    for s in range(_S):
        sp = (s - 1) % _S
        dot = jnp.sum(cur[sp] * cur[s], axis=-1, keepdims=True)
        intra.append(dot / jnp.maximum(na[sp] * na[s], 1e-8))
    return jnp.concatenate(intra, axis=-1), jnp.concatenate(na, axis=-1)


_OHB = np.zeros((_S, _L * _P, _L * _S), np.float32)
for _s in range(_S):
    for _l in range(_L):
        for _p in range(_P):
            _OHB[_s, _l * _P + _p, _l * _S + _PERMS[_p, _s]] = 1.0


def _epi_body(st1_ref, st2_ref, xs_ref, es_ref, ps_ref, xc_ref, oh_ref,
              out_ref):
    nf = jnp.float32(_N)
    st1 = st1_ref[...]
    s_xn = [st1[s:s + 1, :] for s in range(4)]
    s_xf = st1[4:5, :]
    s_ed = [st1[5 + s:6 + s, 0:16] for s in range(4)]
    q_xn = jnp.sum(st1[9:10, 0:1])
    q_xf = jnp.sum(st1[9:10, 1:2])
    q_ed = jnp.sum(st1[9:10, 2:3])
    s2 = jnp.sum(st2_ref[...], axis=0, keepdims=True)
    a_in = [jnp.sum(s2[:, 16 * k:16 * k + 16]) for k in range(4)]
    a_ln = [jnp.sum(s2[:, 64 + 16 * k:80 + 16 * k]) for k in range(4)]
    q_in = jnp.sum(s2[:, 128:192])
    q_ln = jnp.sum(s2[:, 192:256])

    xs = xs_ref[...]
    es = es_ref[...]
    ps = ps_ref[...]
    xc = xc_ref[...]
    ohv = oh_ref[...]
    dn = (((1,), (0,)), ((), ()))

    pps = jnp.concatenate(
        [lax.dot_general(ohv[s], ps, dn) for s in range(4)],
        axis=-1)

    b_in, b_ln = _intra_cols(pps)
    cr_in = sum(b_in[:, k:k + 1] * a_in[k] for k in range(4))
    cr_ln = sum(b_ln[:, k:k + 1] * a_ln[k] for k in range(4))
    t_ang = (q_in - 2.0 * cr_in
             + nf * jnp.sum(b_in * b_in, -1, keepdims=True))
    t_len = (q_ln - 2.0 * cr_ln
             + nf * jnp.sum(b_ln * b_ln, -1, keepdims=True))
    t_sup = jnp.full((_L * _P, 1), q_xn)
    t_edg = jnp.full((_L * _P, 1), q_ed)
    for s in range(4):
        pxs = lax.dot_general(ohv[s], xs, dn)
        ped = lax.dot_general(ohv[s], es, dn)
        t_sup = (t_sup - 2.0 * jnp.sum(pxs * s_xn[s], -1, keepdims=True)
                 + nf * jnp.sum(pxs * pxs, -1, keepdims=True))
        t_edg = (t_edg - 2.0 * jnp.sum(ped * s_ed[s], -1, keepdims=True)
                 + nf * jnp.sum(ped * ped, -1, keepdims=True))

    t_cen8 = (q_xf - 2.0 * jnp.sum(xc * s_xf, -1, keepdims=True)
              + nf * jnp.sum(xc * xc, -1, keepdims=True))

    iota = lax.broadcasted_iota(jnp.int32, (_L * _P, 1), 0)
    lid = iota // _P
    i8 = lax.broadcasted_iota(jnp.int32, (_L, 1), 0)
    big = jnp.float32(3.0e38)
    ot = jnp.zeros((1, _L), jnp.float32)
    oi = lax.broadcasted_iota(jnp.int32, (1, _L), 1)
    for l in range(_L):
        ml = lid == l
        tmin = jnp.min(jnp.where(ml, t_ang, big))
        bidx = jnp.min(jnp.where(ml & (t_ang <= tmin), iota, _L * _P))
        onehot = iota == bidx
        t_len_b = jnp.sum(jnp.where(onehot, t_len, 0.0))
        t_sup_b = jnp.sum(jnp.where(onehot, t_sup, 0.0))
        t_edg_b = jnp.sum(jnp.where(onehot, t_edg, 0.0))
        t_cen = jnp.sum(jnp.where(i8 == l, t_cen8, 0.0))

        sc_ang = _atan_pos(1.0 / tmin)
        sc_len = _atan_pos(1.0 / t_len_b)
        sc_sup = _atan_pos(1.0 / t_sup_b)
        sc_cen = _atan_pos(1.0 / t_cen)
        sc_edg = _atan_pos(1.0 / t_edg_b)

        m = jnp.float32(_M)
        tot = ((sc_len - m) ** 2 + (sc_ang - m) ** 2 + (sc_sup - m) ** 2
               + (sc_cen - m) ** 2 + (sc_edg - m) ** 2)
        sc = _atan_pos(1.0 / tot)
        ot = ot + jnp.where(oi == l, sc, 0.0)
    out_ref[...] = ot


def _epilogue(stats1, stats2, xs, es, ps, xc2):
    return pl.pallas_call(
        _epi_body,
        out_shape=jax.ShapeDtypeStruct((1, _L), jnp.float32),
    )(stats1, stats2, xs, es, ps, xc2, jnp.asarray(_OHB))


def kernel(x_focal, p_focal, x_neighbor, p_neighbor, edge_attr_neighbor,
           x_center, x_support, edge_attr_support, p_support):
    n = x_focal.shape[0]
    pnT = lax.dot_general(jnp.eye(_S * _D, dtype=jnp.float32),
                          p_neighbor.reshape(n, _S * _D),
                          (((1,), (1,)), ((), ()))).reshape(-1)
    pfT = lax.dot_general(jnp.eye(_D, dtype=jnp.float32), p_focal,
                          (((1,), (1,)), ((), ()))).reshape(-1)
    stats2 = _sc_geom(pnT, pfT).reshape(_NWRK, _GEOM)

    stats1 = _tc_stats(x_neighbor, x_focal,
                       edge_attr_neighbor.transpose(1, 2, 0))

    xc2 = x_center.reshape(_L, _ND)
    out = _epilogue(stats1, stats2,
                    x_support.reshape(_L * _S, _ND),
                    edge_attr_support.reshape(_L * _S, _ED),
                    p_support.reshape(_L * _S, _D), xc2)
    return out.reshape(_L)


# speedup vs baseline: 1.9359x; 1.0001x over previous
import math
from itertools import permutations as _permutations

import jax
import jax.numpy as jnp
import numpy as np
from jax import lax
from jax.experimental import pallas as pl
from jax.experimental.pallas import tpu as pltpu
from jax.experimental.pallas import tpu_sc as plsc

_L = 8
_S = 4
_D = 3
_ND = 128
_ED = 16
_N = 10000
_P = 24
_PERMS = np.array(list(_permutations(range(_S))), dtype=np.int32)

_M = math.pi / 2

_GEOM = 256

_BLK = 2000
_G = _N // _BLK

_ATAN_C = (0.9999999227745398, -0.3333223244657235, 0.19974024787565844,
           -0.14047793148813997, 0.10002110154691828, -0.060872867201036907,
           0.02533036269905139, -0.005020633432245819)


def _atan_pos(y):
    big = y > 1.0
    z = jnp.where(big, 1.0 / jnp.maximum(y, 1e-30), y)
    u = z * z
    p = jnp.full_like(u, _ATAN_C[-1])
    for c in _ATAN_C[-2::-1]:
        p = p * u + jnp.float32(c)
    a = z * p
    return jnp.where(big, jnp.float32(_M) - a, a)


def _sc_sqrt(x):
    i = lax.bitcast_convert_type(x, jnp.int32)
    i = jnp.int32(0x5F3759DF) - lax.shift_right_logical(i, 1)
    y = lax.bitcast_convert_type(i, jnp.float32)
    for _ in range(3):
        y = y * (jnp.float32(1.5) - jnp.float32(0.5) * x * y * y)
    return jnp.where(x > 0, x * y, jnp.float32(0.0))


_NGRP = _N // 16
_NWRK = 16
_WG = 40


def _sc_geom_body(pn_hbm, pf_hbm, out_hbm, pnb, pfb, part, sem):
    wid = lax.axis_index("s")
    g0 = (_NGRP * wid) // _NWRK
    g1 = (_NGRP * (wid + 1)) // _NWRK
    d0 = jnp.maximum(g1 - _WG, 0)
    hs = []
    for r in range(12):
        hs.append(pltpu.async_copy(
            pn_hbm.at[pl.ds(r * _N + d0 * 16, _WG * 16)],
            pnb.at[pl.ds(r * _WG * 16, _WG * 16)], sem))
    for r in range(3):
        hs.append(pltpu.async_copy(
            pf_hbm.at[pl.ds(r * _N + d0 * 16, _WG * 16)],
            pfb.at[pl.ds(r * _WG * 16, _WG * 16)], sem))
    zero = jnp.zeros((16,), jnp.float32)
    acc = [[zero] * 4 for _ in range(4)]
    for h in hs:
        h.wait()

    for g in range(_WG):
        gk = d0 + g
        ok = jnp.logical_and(gk >= g0, gk < g1)
        okf = jnp.where(ok, jnp.float32(1.0), jnp.float32(0.0))
        pf_v = [pfb[pl.ds(d * _WG * 16 + 16 * g, 16)] for d in range(3)]
        pe = [pnb[pl.ds((s * 3 + d) * _WG * 16 + 16 * g, 16)] - pf_v[d]
              for s in range(4) for d in range(3)]
        ssq = [pe[3 * s] * pe[3 * s] + pe[3 * s + 1] * pe[3 * s + 1]
               + pe[3 * s + 2] * pe[3 * s + 2] for s in range(4)]
        na = [_sc_sqrt(q) for q in ssq]
        for s in range(4):
            sp = (s - 1) % 4
            dot = (pe[3 * sp] * pe[3 * s]
                   + pe[3 * sp + 1] * pe[3 * s + 1]
                   + pe[3 * sp + 2] * pe[3 * s + 2])
            cosv = dot / jnp.maximum(na[sp] * na[s], jnp.float32(1e-8))
            iv = cosv * okf
            lv = na[s] * okf
            acc[0][s] = acc[0][s] + iv
            acc[1][s] = acc[1][s] + lv
            acc[2][s] = acc[2][s] + iv * iv
            acc[3][s] = acc[3][s] + lv * lv
    for k in range(4):
        for s in range(4):
            part[pl.ds(64 * k + 16 * s, 16)] = acc[k][s]
    pltpu.sync_copy(part, out_hbm.at[pl.ds(wid * _GEOM, _GEOM)])


def _sc_geom(pn1, pf1):
    mesh = plsc.VectorSubcoreMesh(core_axis_name="c", subcore_axis_name="s",
                                  num_cores=1)
    f = pl.kernel(
        _sc_geom_body,
        mesh=mesh,
        out_type=jax.ShapeDtypeStruct((_NWRK * _GEOM,), jnp.float32),
        scratch_types=[
            pltpu.VMEM((12 * _WG * 16,), jnp.float32),
            pltpu.VMEM((3 * _WG * 16,), jnp.float32),
            pltpu.VMEM((_GEOM,), jnp.float32),
            pltpu.SemaphoreType.DMA,
        ],
    )
    return f(pn1, pf1)


def _tc_stats_body(xn_ref, xf_ref, ed_ref, out_ref, a_xn, a_xnq, a_xf,
                   a_xfq):
    i = pl.program_id(0)

    @pl.when(i == 0)
    def _init():
        a_xn[...] = jnp.zeros_like(a_xn)
        a_xnq[...] = jnp.zeros_like(a_xnq)
        a_xf[...] = jnp.zeros_like(a_xf)
        a_xfq[...] = jnp.zeros_like(a_xfq)

    xn = xn_ref[...]
    xf = xf_ref[...]
    ones = jnp.ones((1, _BLK), jnp.float32)
    dnr = (((1,), (0,)), ((), ()))
    a_xn[...] += lax.dot_general(ones, xn, dnr)[0]
    a_xnq[...] += lax.dot_general(ones, xn * xn, dnr)[0]
    a_xf[...] += lax.dot_general(ones, xf, dnr)
    a_xfq[...] += lax.dot_general(ones, xf * xf, dnr)

    @pl.when(i == _G - 1)
    def _fin():
        ed = ed_ref[...]
        a_ed = jnp.sum(ed, axis=2)
        q_ed = jnp.sum(ed * ed)
        li = lax.broadcasted_iota(jnp.int32, (1, 128), 1)
        srow = (jnp.where(li == 0, jnp.sum(a_xnq[...]), 0.0)
                + jnp.where(li == 1, jnp.sum(a_xfq[...]), 0.0)
                + jnp.where(li == 2, q_ed, 0.0))
        zpad = jnp.zeros((1, 112), jnp.float32)
        rows = [a_xn[...], a_xf[...]]
        for s in range(4):
            rows.append(jnp.concatenate([a_ed[s:s + 1, :], zpad], axis=-1))
        rows.append(srow)
        rows.append(jnp.zeros((6, 128), jnp.float32))
        out_ref[...] = jnp.concatenate(rows, axis=0)


def _tc_stats(xn3, xf, edT):
    return pl.pallas_call(
        _tc_stats_body,
        grid=(_G,),
        in_specs=[
            pl.BlockSpec((_BLK, _S, _ND), lambda i: (i, 0, 0)),
            pl.BlockSpec((_BLK, _ND), lambda i: (i, 0)),
            pl.BlockSpec((_S, _ED, _N), lambda i: (0, 0, 0)),
        ],
        out_specs=pl.BlockSpec((16, _ND), lambda i: (0, 0)),
        out_shape=jax.ShapeDtypeStruct((16, _ND), jnp.float32),
        scratch_shapes=[
            pltpu.VMEM((_S, _ND), jnp.float32),
            pltpu.VMEM((_S, _ND), jnp.float32),
            pltpu.VMEM((1, _ND), jnp.float32),
            pltpu.VMEM((1, _ND), jnp.float32),
        ],
    )(xn3, xf, edT)


def _intra_cols(p12):
    cur = [p12[:, 3 * s:3 * s + 3] for s in range(_S)]
    ssq = [jnp.sum(c * c, axis=-1, keepdims=True) for c in cur]
    na = [jnp.sqrt(q) for q in ssq]
    intra = []
    for s in range(_S):
        sp = (s - 1) % _S
        dot = jnp.sum(cur[sp] * cur[s], axis=-1, keepdims=True)
        intra.append(dot / jnp.maximum(na[sp] * na[s], 1e-8))
    return jnp.concatenate(intra, axis=-1), jnp.concatenate(na, axis=-1)


_OHB = np.zeros((_S, _L * _P, _L * _S), np.float32)
for _s in range(_S):
    for _l in range(_L):
        for _p in range(_P):
            _OHB[_s, _l * _P + _p, _l * _S + _PERMS[_p, _s]] = 1.0


def _epi_body(st1_ref, st2_ref, xs_ref, es_ref, ps_ref, xc_ref, oh_ref,
              out_ref):
    nf = jnp.float32(_N)
    st1 = st1_ref[...]
    s_xn = [st1[s:s + 1, :] for s in range(4)]
    s_xf = st1[4:5, :]
    s_ed = [st1[5 + s:6 + s, 0:16] for s in range(4)]
    q_xn = jnp.sum(st1[9:10, 0:1])
    q_xf = jnp.sum(st1[9:10, 1:2])
    q_ed = jnp.sum(st1[9:10, 2:3])
    s2 = jnp.sum(st2_ref[...], axis=0, keepdims=True)
    a_in = [jnp.sum(s2[:, 16 * k:16 * k + 16]) for k in range(4)]
    a_ln = [jnp.sum(s2[:, 64 + 16 * k:80 + 16 * k]) for k in range(4)]
    q_in = jnp.sum(s2[:, 128:192])
    q_ln = jnp.sum(s2[:, 192:256])

    xs = xs_ref[...]
    es = es_ref[...]
    ps = ps_ref[...]
    xc = xc_ref[...]
    ohv = oh_ref[...]
    dn = (((1,), (0,)), ((), ()))

    pps = jnp.concatenate(
        [lax.dot_general(ohv[s], ps, dn) for s in range(4)],
        axis=-1)

    b_in, b_ln = _intra_cols(pps)
    cr_in = sum(b_in[:, k:k + 1] * a_in[k] for k in range(4))
    cr_ln = sum(b_ln[:, k:k + 1] * a_ln[k] for k in range(4))
    t_ang = (q_in - 2.0 * cr_in
             + nf * jnp.sum(b_in * b_in, -1, keepdims=True))
    t_len = (q_ln - 2.0 * cr_ln
             + nf * jnp.sum(b_ln * b_ln, -1, keepdims=True))
    t_sup = jnp.full((_L * _P, 1), q_xn)
    t_edg = jnp.full((_L * _P, 1), q_ed)
    for s in range(4):
        pxs = lax.dot_general(ohv[s], xs, dn)
        ped = lax.dot_general(ohv[s], es, dn)
        t_sup = (t_sup - 2.0 * jnp.sum(pxs * s_xn[s], -1, keepdims=True)
                 + nf * jnp.sum(pxs * pxs, -1, keepdims=True))
        t_edg = (t_edg - 2.0 * jnp.sum(ped * s_ed[s], -1, keepdims=True)
                 + nf * jnp.sum(ped * ped, -1, keepdims=True))

    t_cen8 = (q_xf - 2.0 * jnp.sum(xc * s_xf, -1, keepdims=True)
              + nf * jnp.sum(xc * xc, -1, keepdims=True))

    iota = lax.broadcasted_iota(jnp.int32, (_L * _P, 1), 0)
    lid = iota // _P
    i8 = lax.broadcasted_iota(jnp.int32, (_L, 1), 0)
    big = jnp.float32(3.0e38)
    ot = jnp.zeros((1, _L), jnp.float32)
    oi = lax.broadcasted_iota(jnp.int32, (1, _L), 1)
    for l in range(_L):
        ml = lid == l
        tmin = jnp.min(jnp.where(ml, t_ang, big))
        bidx = jnp.min(jnp.where(ml & (t_ang <= tmin), iota, _L * _P))
        onehot = iota == bidx
        t_len_b = jnp.sum(jnp.where(onehot, t_len, 0.0))
        t_sup_b = jnp.sum(jnp.where(onehot, t_sup, 0.0))
        t_edg_b = jnp.sum(jnp.where(onehot, t_edg, 0.0))
        t_cen = jnp.sum(jnp.where(i8 == l, t_cen8, 0.0))

        sc_ang = _atan_pos(1.0 / tmin)
        sc_len = _atan_pos(1.0 / t_len_b)
        sc_sup = _atan_pos(1.0 / t_sup_b)
        sc_cen = _atan_pos(1.0 / t_cen)
        sc_edg = _atan_pos(1.0 / t_edg_b)

        m = jnp.float32(_M)
        tot = ((sc_len - m) ** 2 + (sc_ang - m) ** 2 + (sc_sup - m) ** 2
               + (sc_cen - m) ** 2 + (sc_edg - m) ** 2)
        sc = _atan_pos(1.0 / tot)
        ot = ot + jnp.where(oi == l, sc, 0.0)
    out_ref[...] = ot


def _epilogue(stats1, stats2, xs, es, ps, xc2):
    return pl.pallas_call(
        _epi_body,
        out_shape=jax.ShapeDtypeStruct((1, _L), jnp.float32),
    )(stats1, stats2, xs, es, ps, xc2, jnp.asarray(_OHB))


def kernel(x_focal, p_focal, x_neighbor, p_neighbor, edge_attr_neighbor,
           x_center, x_support, edge_attr_support, p_support):
    n = x_focal.shape[0]
    pnT = lax.dot_general(jnp.eye(_S * _D, dtype=jnp.float32),
                          p_neighbor.reshape(n, _S * _D),
                          (((1,), (1,)), ((), ()))).reshape(-1)
    pfT = lax.dot_general(jnp.eye(_D, dtype=jnp.float32), p_focal,
                          (((1,), (1,)), ((), ()))).reshape(-1)
    stats2 = _sc_geom(pnT, pfT).reshape(_NWRK, _GEOM)

    stats1 = _tc_stats(x_neighbor, x_focal,
                       edge_attr_neighbor.transpose(1, 2, 0))

    xc2 = x_center.reshape(_L, _ND)
    out = _epilogue(stats1, stats2,
                    x_support.reshape(_L * _S, _ND),
                    edge_attr_support.reshape(_L * _S, _ED),
                    p_support.reshape(_L * _S, _D), xc2)
    return out.reshape(_L)
